# Initial kernel scaffold; baseline (speedup 1.0000x reference)
import jax
import jax.numpy as jnp
from jax.experimental import pallas as pl


def kernel(features_0, features_1, deg_feat, W0, b0, W1, b1, Wc0, bc0, Wc1, bc1, Wl0, bl0, Wl1, bl1, edge_src_0, edge_dst_0, edge_src_1, edge_dst_1, target_node_indices):
    raise NotImplementedError("write your pallas kernel here")


# speedup vs baseline: 3.0760x; 3.0760x over previous
import jax
import jax.numpy as jnp
from jax import lax
from jax.experimental import pallas as pl
from jax.experimental.pallas import tpu as pltpu
from jax.experimental.pallas import tpu_sc as plsc

N = 50000
N0 = 25000
HID = 64
D = 128
OUT = 16
E = 400000
NTGT = 1000

NP = 25088
HALF = NP // 2
HSTRIPE = HALF // 16
ACCROWS = HALF + 16
NT = 50176
ROWB = 512
ROWA = 1000
C = 64
CHUNKS = E // C
TGT_PAD = 1024

_f32 = jnp.float32


def _proj_body(x_ref, dg_ref, w_ref, b_ref, wc_ref, bc_ref, h_ref):
    z = jnp.dot(x_ref[...], w_ref[0], preferred_element_type=_f32) + b_ref[0]
    zc = jnp.dot(dg_ref[...], wc_ref[0], preferred_element_type=_f32) + bc_ref[0]
    h_ref[:, :64] = jnp.maximum(z, 0.9 * z)
    h_ref[:, 64:] = jnp.maximum(zc, 0.9 * zc)


def _projections(x, degf, wstack, bstack, wcstack, bcstack):
    nblk = N // ROWA
    half = nblk // 2
    return pl.pallas_call(
        _proj_body,
        grid=(nblk,),
        in_specs=[
            pl.BlockSpec((ROWA, 128), lambda i: (i, 0)),
            pl.BlockSpec((ROWA, 64), lambda i: (i, 0)),
            pl.BlockSpec((1, 128, 64), lambda i: (i // half, 0, 0)),
            pl.BlockSpec((1, 1, 64), lambda i: (i // half, 0, 0)),
            pl.BlockSpec((1, 64, 64), lambda i: (i // half, 0, 0)),
            pl.BlockSpec((1, 1, 64), lambda i: (i // half, 0, 0)),
        ],
        out_specs=pl.BlockSpec((ROWA, 128), lambda i: (i, 0)),
        out_shape=jax.ShapeDtypeStruct((N, 128), _f32),
    )(x, degf, wstack, bstack, wcstack, bcstack)


def _make_seg_kernel(with_deg):
    mesh = plsc.VectorSubcoreMesh(
        core_axis_name="c", subcore_axis_name="s", num_cores=2, num_subcores=16)

    if with_deg:
        out_type = [
            jax.ShapeDtypeStruct((2, NP, 128), _f32),
            jax.ShapeDtypeStruct((2, NP, 16), _f32),
        ]
    else:
        out_type = [jax.ShapeDtypeStruct((2, NP, 128), _f32)]

    scratch = [
        pltpu.VMEM((C,), jnp.int32),
        pltpu.VMEM((C,), jnp.int32),
        pltpu.VMEM((C,), jnp.int32),
        pltpu.VMEM((C,), jnp.int32),
        pltpu.VMEM((C,), jnp.int32),
        pltpu.VMEM((C,), jnp.int32),
        pltpu.VMEM((C, 128), _f32),
        pltpu.VMEM((C, 128), _f32),
        pltpu.SemaphoreType.DMA,
        pltpu.SemaphoreType.DMA,
        pltpu.VMEM((C,), _f32),
        pltpu.VMEM((16, 128), _f32),
        pltpu.VMEM((16,), _f32),
        pltpu.VMEM((16,), _f32),
        pltpu.VMEM((16, 16), _f32),
        pltpu.VMEM((16, 128), _f32),
        pltpu.VMEM_SHARED((ACCROWS, 128), _f32),
        pltpu.VMEM_SHARED((ACCROWS,), _f32),
    ]

    def body(table_h, s0_h, d0_h, s1_h, d1_h, *rest):
        if with_deg:
            agg_h, rdeg_h = rest[0], rest[1]
            rest = rest[2:]
        else:
            rdeg_in, agg_h = rest[0], rest[1]
            rest = rest[2:]
        (srcva, srcvb, dstva, dstvb, dstv2a, dstv2b, rowsa, rowsb,
         sema, semb, onesv, zb, zd, degv, rd16, accv, acc, dega) = rest

        c = lax.axis_index("c")
        s = lax.axis_index("s")
        base_dst = c * HALF
        dummy = HALF + s
        out0 = c * HALF + s * HSTRIPE
        nb = (CHUNKS + 15 - s) // 16

        for r in range(16):
            for j in range(8):
                zb[r, pl.ds(16 * j, 16)] = jnp.zeros((16,), _f32)
        for j in range(C // 16):
            onesv[pl.ds(16 * j, 16)] = jnp.ones((16,), _f32)
        zd[...] = jnp.zeros((16,), _f32)

        for p in range(2):
            src_h = s0_h if p == 0 else s1_h
            dst_h = d0_h if p == 0 else d1_h

            def zbody(k, _):
                pltpu.sync_copy(zb, acc.at[pl.ds(s * HSTRIPE + 16 * k, 16), :])
                pltpu.sync_copy(zd, dega.at[pl.ds(s * HSTRIPE + 16 * k, 16)])
                return 0
            lax.fori_loop(0, HSTRIPE // 16, zbody, 0)
            plsc.subcore_barrier()

            def start(k, srcv, dstv, dstv2, rows, sem):
                @pl.when(k < nb)
                def _():
                    base = (s + 16 * k) * C
                    pltpu.sync_copy(src_h.at[pl.ds(base, C)], srcv)
                    pltpu.sync_copy(dst_h.at[pl.ds(base, C)], dstv)
                    pltpu.async_copy(table_h.at[srcv], rows, sem)
                    for g in range(C // 16):
                        dv = dstv[pl.ds(16 * g, 16)]
                        lv = dv - base_dst
                        m = (lv >= 0) & (lv < HALF)
                        dstv2[pl.ds(16 * g, 16)] = jnp.where(
                            m, lv, jnp.broadcast_to(dummy, (16,)))

            def drain(k, srcv, dstv, dstv2, rows, sem):
                @pl.when(k < nb)
                def _():
                    pltpu.make_async_copy(table_h.at[srcv], rows, sem).wait()
                    pltpu.sync_copy(rows, acc.at[dstv2], add=True)
                    if with_deg:
                        pltpu.sync_copy(onesv, dega.at[dstv2], add=True)

            start(0, srcva, dstva, dstv2a, rowsa, sema)
            start(1, srcvb, dstvb, dstv2b, rowsb, semb)

            def ebody(i, _):
                k0 = 2 * i
                drain(k0, srcva, dstva, dstv2a, rowsa, sema)
                start(k0 + 2, srcva, dstva, dstv2a, rowsa, sema)
                drain(k0 + 1, srcvb, dstvb, dstv2b, rowsb, semb)
                start(k0 + 3, srcvb, dstvb, dstv2b, rowsb, semb)
                return 0
            lax.fori_loop(0, (CHUNKS // 16 + 2) // 2, ebody, 0)
            plsc.subcore_barrier()

            if with_deg:
                pltpu.sync_copy(acc.at[pl.ds(s * HSTRIPE, HSTRIPE), :],
                                agg_h.at[p].at[pl.ds(out0, HSTRIPE), :])

                def nbody(k, _):
                    r0 = s * HSTRIPE + 16 * k
                    pltpu.sync_copy(dega.at[pl.ds(r0, 16)], degv)
                    rdv = 1.0 / jnp.maximum(degv[...], 1.0)
                    for i in range(16):
                        rd16[i, :] = jnp.broadcast_to(rdv[i], (16,))
                    pltpu.sync_copy(
                        rd16, rdeg_h.at[p].at[pl.ds(out0 + 16 * k, 16), :])
                    return 0
                lax.fori_loop(0, HSTRIPE // 16, nbody, 0)
            else:
                def nbody(k, _):
                    r0 = s * HSTRIPE + 16 * k
                    pltpu.sync_copy(acc.at[pl.ds(r0, 16), :], accv)
                    pltpu.sync_copy(
                        rdeg_in.at[p].at[pl.ds(out0 + 16 * k, 16), :], rd16)
                    for i in range(16):
                        srow = rd16[i, :]
                        for j in range(8):
                            accv[i, pl.ds(16 * j, 16)] = (
                                accv[i, pl.ds(16 * j, 16)] * srow)
                    pltpu.sync_copy(
                        accv, agg_h.at[p].at[pl.ds(out0 + 16 * k, 16), :])
                    return 0
                lax.fori_loop(0, HSTRIPE // 16, nbody, 0)
            plsc.subcore_barrier()

    return pl.kernel(body, out_type=out_type, mesh=mesh, scratch_types=scratch)


_seg_deg = _make_seg_kernel(True)
_seg_norm = _make_seg_kernel(False)


def _combine0_body(agg_ref, rd_ref, w_ref, b_ref, h_ref):
    i = pl.program_id(0)
    valid = (i < NP // ROWB).astype(_f32)
    ctr = (0.5 * valid) * (agg_ref[0] * rd_ref[0][:, :1]
                           + agg_ref[1] * rd_ref[1][:, :1])
    z = jnp.dot(ctr, w_ref[...], preferred_element_type=_f32) + b_ref[...]
    h_ref[...] = jnp.where(z > 0, z, jnp.exp(jnp.minimum(z, 0.0)) - 1.0)


def _combine0(agg, rdeg, wl0, bl0):
    nblk = NT // ROWB
    clamp = NP // ROWB - 1
    return pl.pallas_call(
        _combine0_body,
        grid=(nblk,),
        in_specs=[
            pl.BlockSpec((2, ROWB, 128), lambda i: (0, jnp.minimum(i, clamp), 0)),
            pl.BlockSpec((2, ROWB, 16), lambda i: (0, jnp.minimum(i, clamp), 0)),
            pl.BlockSpec((128, 128), lambda i: (0, 0)),
            pl.BlockSpec((1, 128), lambda i: (0, 0)),
        ],
        out_specs=pl.BlockSpec((ROWB, 128), lambda i: (i, 0)),
        out_shape=jax.ShapeDtypeStruct((NT, 128), _f32),
    )(agg, rdeg, wl0, bl0)


def _make_tgt_gather():
    mesh = plsc.VectorSubcoreMesh(
        core_axis_name="c", subcore_axis_name="s", num_cores=2, num_subcores=16)
    per_w = TGT_PAD // 32

    out_type = [jax.ShapeDtypeStruct((2, TGT_PAD, 128), _f32)]
    scratch = [
        pltpu.VMEM((per_w,), jnp.int32),
        pltpu.VMEM((per_w, 128), _f32),
        pltpu.SemaphoreType.DMA,
    ]

    def body(n1_h, tgt_h, g_h, idxv, buf, sem):
        c = lax.axis_index("c")
        s = lax.axis_index("s")
        base = (s * 2 + c) * per_w
        pltpu.sync_copy(tgt_h.at[pl.ds(base, per_w)], idxv)
        for p in range(2):
            pltpu.async_copy(n1_h.at[p].at[idxv], buf, sem).wait()
            pltpu.sync_copy(buf, g_h.at[p].at[pl.ds(base, per_w), :])

    return pl.kernel(body, out_type=out_type, mesh=mesh, scratch_types=scratch)


_tgt_gather = _make_tgt_gather()


def _final_body(g_ref, w_ref, b_ref, hn_ref, log_ref):
    hn = 0.5 * (g_ref[0] + g_ref[1])
    hn_ref[...] = hn
    log_ref[...] = (jnp.dot(hn, w_ref[...], preferred_element_type=_f32)
                    + b_ref[...])


def _final(g, wl1p, bl1p):
    return pl.pallas_call(
        _final_body,
        out_shape=[
            jax.ShapeDtypeStruct((TGT_PAD, 128), _f32),
            jax.ShapeDtypeStruct((TGT_PAD, 128), _f32),
        ],
    )(g, wl1p, bl1p)


def kernel(features_0, features_1, deg_feat, W0, b0, W1, b1, Wc0, bc0, Wc1,
           bc1, Wl0, bl0, Wl1, bl1, edge_src_0, edge_dst_0, edge_src_1,
           edge_dst_1, target_node_indices):
    x = jnp.concatenate(
        [features_0, jnp.pad(features_1, ((0, 0), (0, 64)))], axis=0)
    wstack = jnp.stack([W0, jnp.pad(W1, ((0, 64), (0, 0)))])
    bstack = jnp.stack([b0, b1]).reshape(2, 1, HID)
    wcstack = jnp.stack([Wc0, Wc1])
    bcstack = jnp.stack([bc0, bc1]).reshape(2, 1, HID)
    s0 = edge_src_0.astype(jnp.int32)
    d0 = edge_dst_0.astype(jnp.int32)
    s1 = edge_src_1.astype(jnp.int32)
    d1 = edge_dst_1.astype(jnp.int32)
    tgt = jnp.concatenate([
        target_node_indices.astype(jnp.int32),
        jnp.zeros((TGT_PAD - NTGT,), jnp.int32)])
    wl1p = jnp.pad(Wl1, ((0, 0), (0, 128 - OUT)))
    bl1p = jnp.pad(bl1, (0, 128 - OUT)).reshape(1, 128)
    bl0r = bl0.reshape(1, 128)

    h = _projections(x, deg_feat, wstack, bstack, wcstack, bcstack)

    agg0, rdeg = _seg_deg(h, s0, d0, s1, d1)
    h1 = _combine0(agg0, rdeg, Wl0, bl0r)

    n1 = _seg_norm(h1, s0, d0, s1, d1, rdeg)
    if isinstance(n1, (tuple, list)):
        n1 = n1[0]

    g = _tgt_gather(n1, tgt)
    if isinstance(g, (tuple, list)):
        g = g[0]
    hn, logp = _final(g, wl1p, bl1p)

    logits = logp[:NTGT, :OUT]
    hnew = hn[:NTGT]
    return logits, hnew


# speedup vs baseline: 4.0672x; 1.3223x over previous
import jax
import jax.numpy as jnp
from jax import lax
from jax.experimental import pallas as pl
from jax.experimental.pallas import tpu as pltpu
from jax.experimental.pallas import tpu_sc as plsc

N = 50000
N0 = 25000
HID = 64
D = 128
OUT = 16
E = 400000
NTGT = 1000

NP = 25088
HALF = NP // 2
HSTRIPE = HALF // 16
ACCROWS = HALF + 16
NT = 50176
ROWB = 512
ROWA = 1000
C = 64
SUP = 512
SUBS = SUP // C
EPAD = 400384
NSUP = EPAD // SUP
NSUP_DEG = EPAD // 1024
TGT_PAD = 1024

_f32 = jnp.float32


def _proj_body(x_ref, dg_ref, w_ref, b_ref, wc_ref, bc_ref, h_ref):
    z = jnp.dot(x_ref[...], w_ref[0], preferred_element_type=_f32) + b_ref[0]
    zc = jnp.dot(dg_ref[...], wc_ref[0], preferred_element_type=_f32) + bc_ref[0]
    h_ref[:, :64] = jnp.maximum(z, 0.9 * z)
    h_ref[:, 64:] = jnp.maximum(zc, 0.9 * zc)


def _projections(x, degf, wstack, bstack, wcstack, bcstack):
    nblk = N // ROWA
    half = nblk // 2
    return pl.pallas_call(
        _proj_body,
        grid=(nblk,),
        in_specs=[
            pl.BlockSpec((ROWA, 128), lambda i: (i, 0)),
            pl.BlockSpec((ROWA, 64), lambda i: (i, 0)),
            pl.BlockSpec((1, 128, 64), lambda i: (i // half, 0, 0)),
            pl.BlockSpec((1, 1, 64), lambda i: (i // half, 0, 0)),
            pl.BlockSpec((1, 64, 64), lambda i: (i // half, 0, 0)),
            pl.BlockSpec((1, 1, 64), lambda i: (i // half, 0, 0)),
        ],
        out_specs=pl.BlockSpec((ROWA, 128), lambda i: (i, 0)),
        out_shape=jax.ShapeDtypeStruct((N, 128), _f32),
    )(x, degf, wstack, bstack, wcstack, bcstack)


def _make_deg_kernel():
    mesh = plsc.VectorSubcoreMesh(
        core_axis_name="c", subcore_axis_name="s", num_cores=2, num_subcores=16)
    out_type = [jax.ShapeDtypeStruct((2, NP, 16), _f32)]
    scratch = [
        pltpu.VMEM((16, C), jnp.int32),
        pltpu.VMEM((C,), _f32),
        pltpu.VMEM((16,), _f32),
        pltpu.VMEM((16,), _f32),
        pltpu.VMEM((16, 16), _f32),
        pltpu.SemaphoreType.DMA,
        pltpu.VMEM_SHARED((NP,), _f32),
    ]
    stripe = NP // 16

    def body(d0_h, d1_h, rdeg_h, dst2d, onesv, zd, degv, rd16, sem, dega):
        c = lax.axis_index("c")
        s = lax.axis_index("s")
        nm = (NSUP_DEG - s + 15) // 16

        for j in range(C // 16):
            onesv[pl.ds(16 * j, 16)] = jnp.ones((16,), _f32)
        zd[...] = jnp.zeros((16,), _f32)

        def zbody(k, _):
            pltpu.sync_copy(zd, dega.at[pl.ds(s * stripe + 16 * k, 16)])
            return 0
        lax.fori_loop(0, stripe // 16, zbody, 0)
        plsc.subcore_barrier()

        def run(dst_h):
            def sbody(m, _):
                g = s + 16 * m
                pltpu.sync_copy(dst_h.at[pl.ds(16 * g, 16), :], dst2d)
                for j in range(16):
                    pltpu.async_copy(onesv, dega.at[dst2d.at[j]], sem, add=True)
                for j in range(16):
                    pltpu.make_async_copy(
                        onesv, dega.at[dst2d.at[j]], sem).wait()
                return 0
            lax.fori_loop(0, nm, sbody, 0)

        @pl.when(c == 0)
        def _():
            run(d0_h)

        @pl.when(c == 1)
        def _():
            run(d1_h)

        plsc.subcore_barrier()

        def nbody(k, _):
            r0 = s * stripe + 16 * k
            pltpu.sync_copy(dega.at[pl.ds(r0, 16)], degv)
            rdv = 1.0 / jnp.maximum(degv[...], 1.0)
            for i in range(16):
                rd16[i, :] = jnp.broadcast_to(rdv[i], (16,))
            pltpu.sync_copy(rd16, rdeg_h.at[c].at[pl.ds(r0, 16), :])
            return 0
        lax.fori_loop(0, stripe // 16, nbody, 0)

    return pl.kernel(body, out_type=out_type, mesh=mesh, scratch_types=scratch)


_deg_kernel = _make_deg_kernel()


def _make_seg_kernel(norm):
    mesh = plsc.VectorSubcoreMesh(
        core_axis_name="c", subcore_axis_name="s", num_cores=2, num_subcores=16)

    out_type = [jax.ShapeDtypeStruct((2, NP, 128), _f32)]

    scratch = [
        pltpu.VMEM((SUBS, C), jnp.int32),
        pltpu.VMEM((SUBS, C), jnp.int32),
        pltpu.VMEM((C, 128), _f32),
        pltpu.VMEM((C, 128), _f32),
        pltpu.VMEM((C, 128), _f32),
        pltpu.SemaphoreType.DMA,
        pltpu.SemaphoreType.DMA,
        pltpu.SemaphoreType.DMA,
        pltpu.SemaphoreType.DMA,
        pltpu.SemaphoreType.DMA,
        pltpu.SemaphoreType.DMA,
        pltpu.VMEM_SHARED((ACCROWS, 128), _f32),
    ]
    if norm:
        scratch += [
            pltpu.VMEM((8, 128), _f32),
            pltpu.VMEM((8, 16), _f32),
        ]

    def body(table_h, s0_h, d0_h, s1_h, d1_h, *rest):
        if norm:
            rdeg_in = rest[0]
            agg_h = rest[1]
            (srcbig, dstbig, rows0, rows1, rows2, semg0, semg1, semg2,
             sems0, sems1, sems2, acc, accv, rd16) = rest[2:]
        else:
            agg_h = rest[0]
            (srcbig, dstbig, rows0, rows1, rows2, semg0, semg1, semg2,
             sems0, sems1, sems2, acc) = rest[1:]

        rows = [rows0, rows1, rows2]
        semg = [semg0, semg1, semg2]
        sems = [sems0, sems1, sems2]

        c = lax.axis_index("c")
        s = lax.axis_index("s")
        base_dst = c * HALF
        dummy = HALF + s
        out0 = c * HALF + s * HSTRIPE
        nm = (NSUP - s + 15) // 16

        for p in range(2):
            src_h = s0_h if p == 0 else s1_h
            dst_h = d0_h if p == 0 else d1_h

            for r in range(8):
                for j in range(8):
                    rows0[r, pl.ds(16 * j, 16)] = jnp.zeros((16,), _f32)

            def zbody(k, _):
                pltpu.sync_copy(rows0.at[pl.ds(0, 8), :],
                                acc.at[pl.ds(s * HSTRIPE + 8 * k, 8), :])
                return 0
            lax.fori_loop(0, HSTRIPE // 8, zbody, 0)
            plsc.subcore_barrier()

            def startg(j, b):
                pltpu.async_copy(
                    table_h.at[srcbig.at[j]], rows[b], semg[b])

            def waitg(j, b):
                pltpu.make_async_copy(
                    table_h.at[srcbig.at[j]], rows[b], semg[b]).wait()

            def starts(j, b):
                pltpu.async_copy(
                    rows[b], acc.at[dstbig.at[j]], sems[b], add=True)

            def waits(j, b):
                pltpu.make_async_copy(
                    rows[b], acc.at[dstbig.at[j]], sems[b]).wait()

            def sbody(m, _):
                g = s + 16 * m
                pltpu.sync_copy(src_h.at[pl.ds(SUBS * g, SUBS), :], srcbig)
                pltpu.sync_copy(dst_h.at[pl.ds(SUBS * g, SUBS), :], dstbig)
                startg(0, 0)
                for j in range(SUBS):
                    for q in range(C // 16):
                        dv = dstbig[j, pl.ds(16 * q, 16)]
                        lv = dv - base_dst
                        msk = (lv >= 0) & (lv < HALF)
                        dstbig[j, pl.ds(16 * q, 16)] = jnp.where(
                            msk, lv, jnp.broadcast_to(dummy, (16,)))
                startg(1, 1)
                for j in range(SUBS):
                    b = j % 3
                    waitg(j, b)
                    starts(j, b)
                    if j + 2 < SUBS:
                        if j >= 1:
                            waits(j - 1, (j - 1) % 3)
                        startg(j + 2, (j + 2) % 3)
                waits(SUBS - 2, (SUBS - 2) % 3)
                waits(SUBS - 1, (SUBS - 1) % 3)
                return 0
            lax.fori_loop(0, nm, sbody, 0)
            plsc.subcore_barrier()

            if not norm:
                pltpu.sync_copy(acc.at[pl.ds(s * HSTRIPE, HSTRIPE), :],
                                agg_h.at[p].at[pl.ds(out0, HSTRIPE), :])
            else:
                def nbody(k, _):
                    r0 = s * HSTRIPE + 8 * k
                    pltpu.sync_copy(acc.at[pl.ds(r0, 8), :], accv)
                    pltpu.sync_copy(
                        rdeg_in.at[p].at[pl.ds(out0 + 8 * k, 8), :], rd16)
                    for i in range(8):
                        srow = rd16[i, :]
                        for j in range(8):
                            accv[i, pl.ds(16 * j, 16)] = (
                                accv[i, pl.ds(16 * j, 16)] * srow)
                    pltpu.sync_copy(
                        accv, agg_h.at[p].at[pl.ds(out0 + 8 * k, 8), :])
                    return 0
                lax.fori_loop(0, HSTRIPE // 8, nbody, 0)
            plsc.subcore_barrier()

    return pl.kernel(body, out_type=out_type, mesh=mesh, scratch_types=scratch)


_seg_raw = _make_seg_kernel(False)
_seg_norm = _make_seg_kernel(True)


def _combine0_body(agg_ref, rd_ref, w_ref, b_ref, h_ref):
    i = pl.program_id(0)
    valid = (i < NP // ROWB).astype(_f32)
    row = lax.broadcasted_iota(jnp.int32, (ROWB, 1), 0) + i * ROWB
    live = (row < N0).astype(_f32) * valid
    ctr = (0.5 * live) * (agg_ref[0] * rd_ref[0][:, :1]
                          + agg_ref[1] * rd_ref[1][:, :1])
    z = jnp.dot(ctr, w_ref[...], preferred_element_type=_f32) + b_ref[...]
    h_ref[...] = jnp.where(z > 0, z, jnp.exp(jnp.minimum(z, 0.0)) - 1.0)


def _combine0(agg, rdeg, wl0, bl0):
    nblk = NT // ROWB
    clamp = NP // ROWB - 1
    return pl.pallas_call(
        _combine0_body,
        grid=(nblk,),
        in_specs=[
            pl.BlockSpec((2, ROWB, 128), lambda i: (0, jnp.minimum(i, clamp), 0)),
            pl.BlockSpec((2, ROWB, 16), lambda i: (0, jnp.minimum(i, clamp), 0)),
            pl.BlockSpec((128, 128), lambda i: (0, 0)),
            pl.BlockSpec((1, 128), lambda i: (0, 0)),
        ],
        out_specs=pl.BlockSpec((ROWB, 128), lambda i: (i, 0)),
        out_shape=jax.ShapeDtypeStruct((NT, 128), _f32),
    )(agg, rdeg, wl0, bl0)


def _make_tgt_gather():
    mesh = plsc.VectorSubcoreMesh(
        core_axis_name="c", subcore_axis_name="s", num_cores=2, num_subcores=16)
    per_w = TGT_PAD // 32

    out_type = [jax.ShapeDtypeStruct((2, TGT_PAD, 128), _f32)]
    scratch = [
        pltpu.VMEM((per_w,), jnp.int32),
        pltpu.VMEM((per_w, 128), _f32),
        pltpu.SemaphoreType.DMA,
    ]

    def body(n1_h, tgt_h, g_h, idxv, buf, sem):
        c = lax.axis_index("c")
        s = lax.axis_index("s")
        base = (s * 2 + c) * per_w
        pltpu.sync_copy(tgt_h.at[pl.ds(base, per_w)], idxv)
        for p in range(2):
            pltpu.async_copy(n1_h.at[p].at[idxv], buf, sem).wait()
            pltpu.sync_copy(buf, g_h.at[p].at[pl.ds(base, per_w), :])

    return pl.kernel(body, out_type=out_type, mesh=mesh, scratch_types=scratch)


_tgt_gather = _make_tgt_gather()


def _final_body(g_ref, w_ref, b_ref, hn_ref, log_ref):
    hn = 0.5 * (g_ref[0] + g_ref[1])
    hn_ref[...] = hn
    log_ref[...] = (jnp.dot(hn, w_ref[...], preferred_element_type=_f32)
                    + b_ref[...])


def _final(g, wl1p, bl1p):
    return pl.pallas_call(
        _final_body,
        out_shape=[
            jax.ShapeDtypeStruct((TGT_PAD, 128), _f32),
            jax.ShapeDtypeStruct((TGT_PAD, 128), _f32),
        ],
    )(g, wl1p, bl1p)


def kernel(features_0, features_1, deg_feat, W0, b0, W1, b1, Wc0, bc0, Wc1,
           bc1, Wl0, bl0, Wl1, bl1, edge_src_0, edge_dst_0, edge_src_1,
           edge_dst_1, target_node_indices):
    x = jnp.concatenate(
        [features_0, jnp.pad(features_1, ((0, 0), (0, 64)))], axis=0)
    wstack = jnp.stack([W0, jnp.pad(W1, ((0, 64), (0, 0)))])
    bstack = jnp.stack([b0, b1]).reshape(2, 1, HID)
    wcstack = jnp.stack([Wc0, Wc1])
    bcstack = jnp.stack([bc0, bc1]).reshape(2, 1, HID)

    pad_s = jnp.zeros((EPAD - E,), jnp.int32)
    pad_d = jnp.full((EPAD - E,), N0, jnp.int32)

    def prep(a, pad):
        return jnp.concatenate([a.astype(jnp.int32), pad]).reshape(-1, C)

    s0 = prep(edge_src_0, pad_s)
    d0 = prep(edge_dst_0, pad_d)
    s1 = prep(edge_src_1, pad_s)
    d1 = prep(edge_dst_1, pad_d)
    tgt = jnp.concatenate([
        target_node_indices.astype(jnp.int32),
        jnp.zeros((TGT_PAD - NTGT,), jnp.int32)])
    wl1p = jnp.pad(Wl1, ((0, 0), (0, 128 - OUT)))
    bl1p = jnp.pad(bl1, (0, 128 - OUT)).reshape(1, 128)
    bl0r = bl0.reshape(1, 128)

    h = _projections(x, deg_feat, wstack, bstack, wcstack, bcstack)
    rdeg = _deg_kernel(d0, d1)
    if isinstance(rdeg, (tuple, list)):
        rdeg = rdeg[0]

    agg0 = _seg_raw(h, s0, d0, s1, d1)
    if isinstance(agg0, (tuple, list)):
        agg0 = agg0[0]
    h1 = _combine0(agg0, rdeg, Wl0, bl0r)

    n1 = _seg_norm(h1, s0, d0, s1, d1, rdeg)
    if isinstance(n1, (tuple, list)):
        n1 = n1[0]

    g = _tgt_gather(n1, tgt)
    if isinstance(g, (tuple, list)):
        g = g[0]
    hn, logp = _final(g, wl1p, bl1p)

    logits = logp[:NTGT, :OUT]
    hnew = hn[:NTGT]
    return logits, hnew
